# Initial kernel scaffold; baseline (speedup 1.0000x reference)
import jax
import jax.numpy as jnp
from jax.experimental import pallas as pl


def kernel(toks, tok_emb, W, b, stdev, noise):
    raise NotImplementedError("write your pallas kernel here")


# speedup vs baseline: 2.2836x; 2.2836x over previous
import functools

import jax
import jax.numpy as jnp
from jax import lax
from jax.experimental import pallas as pl
from jax.experimental.pallas import tpu as pltpu
from jax.experimental.pallas import tpu_sc as plsc

_NC = 2
_NS = 16
_LANES = 16
_MAX_IDX = 128


def _make_pooling_kernel(N, T, EMB):
    NW = _NC * _NS
    assert N % NW == 0, N
    SPW = N // NW
    assert SPW % 2 == 0, SPW
    assert EMB % _LANES == 0, EMB
    assert T % 8 == 0, T

    chunks = []
    off = 0
    while off < T:
        c = min(_MAX_IDX, T - off)
        chunks.append((off, c))
        off += c

    mesh = plsc.VectorSubcoreMesh(
        core_axis_name="c", subcore_axis_name="s",
        num_cores=_NC, num_subcores=_NS)

    def body(toks_hbm, table_hbm, out_hbm,
             idx_v, rows_a, rows_b, pooled_v, sem_a, sem_b):
        wid = lax.axis_index("s") * _NC + lax.axis_index("c")
        tok_base = pl.multiple_of(wid * (SPW * T), 8)
        pltpu.sync_copy(toks_hbm.at[pl.ds(tok_base, SPW * T)], idx_v)

        def copies(s, rows, sem):
            res = []
            for (o, c) in chunks:
                src = table_hbm.at[idx_v.at[pl.ds(pl.multiple_of(s * T + o, 8), c)]]
                res.append(pltpu.make_async_copy(src, rows.at[pl.ds(o, c)], sem))
            return res

        def start(s, rows, sem):
            for cp in copies(s, rows, sem):
                cp.start()

        def wait(s, rows, sem):
            for cp in copies(s, rows, sem):
                cp.wait()

        def reduce_store(s, rows):
            ngrp = EMB // _LANES

            def rbody(t, acc):
                return tuple(acc[g] + rows[t, pl.ds(g * _LANES, _LANES)]
                             for g in range(ngrp))

            zero = tuple(jnp.zeros((_LANES,), jnp.float32) for _ in range(ngrp))
            acc = lax.fori_loop(0, T, rbody, zero, unroll=8)
            for g in range(ngrp):
                pooled_v[s, pl.ds(g * _LANES, _LANES)] = acc[g]

        start(0, rows_a, sem_a)

        def pair(p, carry):
            s0 = p * 2
            start(s0 + 1, rows_b, sem_b)
            wait(s0, rows_a, sem_a)
            reduce_store(s0, rows_a)

            @pl.when(s0 + 2 < SPW)
            def _():
                start(s0 + 2, rows_a, sem_a)

            wait(s0 + 1, rows_b, sem_b)
            reduce_store(s0 + 1, rows_b)
            return carry

        lax.fori_loop(0, SPW // 2, pair, 0)

        out_base = pl.multiple_of(wid * SPW, 8)
        pltpu.sync_copy(pooled_v, out_hbm.at[pl.ds(out_base, SPW)])

    return pl.kernel(
        body,
        out_type=jax.ShapeDtypeStruct((N, EMB), jnp.float32),
        mesh=mesh,
        compiler_params=pltpu.CompilerParams(use_tc_tiling_on_sc=False),
        scratch_types=[
            pltpu.VMEM((SPW * T,), jnp.int32),
            pltpu.VMEM((T, EMB), jnp.float32),
            pltpu.VMEM((T, EMB), jnp.float32),
            pltpu.VMEM((SPW, EMB), jnp.float32),
            pltpu.SemaphoreType.DMA,
            pltpu.SemaphoreType.DMA,
        ],
    )


def _head_body(ps_ref, w_ref, b_ref, stdev_ref, noise_ref, out_ref, *, T):
    pooled = ps_ref[...] * (1.0 / T)
    logits = jnp.dot(pooled, w_ref[...], preferred_element_type=jnp.float32)
    logits = logits + b_ref[...] + stdev_ref[0, 0] * noise_ref[...]
    out_ref[...] = jax.nn.log_softmax(logits, axis=-1)


def kernel(toks, tok_emb, W, b, stdev, noise):
    N, T = toks.shape
    EMB = tok_emb.shape[1]
    NCLS = W.shape[1]
    toks_flat = toks.reshape(-1).astype(jnp.int32)
    pooled_sum = _make_pooling_kernel(N, T, EMB)(toks_flat, tok_emb)
    head = pl.pallas_call(
        functools.partial(_head_body, T=T),
        out_shape=jax.ShapeDtypeStruct((N, NCLS), jnp.float32),
    )
    return head(pooled_sum, W, b.reshape(1, NCLS),
                stdev.reshape(1, 1), noise)


# speedup vs baseline: 2.5676x; 1.1244x over previous
import functools

import jax
import jax.numpy as jnp
from jax import lax
from jax.experimental import pallas as pl
from jax.experimental.pallas import tpu as pltpu
from jax.experimental.pallas import tpu_sc as plsc

_NC = 2
_NS = 16
_LANES = 16
_MAX_IDX = 128
_CHUNK = 2048


def _relayout_body(src_ref, toks_ref, out_ref, idx_ref, *, EMB):
    x = src_ref[...]
    G = _CHUNK // 4
    parts = [jnp.swapaxes(x[:, G * q:G * (q + 1)], 0, 1) for q in range(4)]
    out_ref[...] = jnp.concatenate(parts, axis=1)
    t = toks_ref[...]
    rem = jnp.bitwise_and(t, _CHUNK - 1)
    g = (jnp.bitwise_and(t, jnp.int32(~(_CHUNK - 1))) +
         (jnp.bitwise_and(rem, G - 1) << 2) + (rem // G))
    idx_ref[...] = g


def _make_relayout(V, EMB, NT):
    grid = (V + _CHUNK - 1) // _CHUNK
    rows_out = grid * (_CHUNK // 4)
    assert grid * _CHUNK >= NT
    return pl.pallas_call(
        functools.partial(_relayout_body, EMB=EMB),
        grid=(grid,),
        in_specs=[
            pl.BlockSpec((EMB, _CHUNK), lambda i: (0, i)),
            pl.BlockSpec((_CHUNK,), lambda i: (i,)),
        ],
        out_specs=[
            pl.BlockSpec((_CHUNK // 4, 4 * EMB), lambda i: (i, 0)),
            pl.BlockSpec((_CHUNK,), lambda i: (i,)),
        ],
        out_shape=[
            jax.ShapeDtypeStruct((rows_out, 4 * EMB), jnp.float32),
            jax.ShapeDtypeStruct((grid * _CHUNK,), jnp.int32),
        ],
    )


def _make_pooling_kernel(N, T, EMB, VROWS):
    NW = _NC * _NS
    assert N % NW == 0, N
    SPW = N // NW
    assert SPW % 2 == 0, SPW
    assert EMB % _LANES == 0, EMB
    assert T % 8 == 0, T

    chunks = []
    off = 0
    while off < T:
        c = min(_MAX_IDX, T - off)
        chunks.append((off, c))
        off += c

    mesh = plsc.VectorSubcoreMesh(
        core_axis_name="c", subcore_axis_name="s",
        num_cores=_NC, num_subcores=_NS)

    def body(toks_hbm, table_hbm, out_hbm,
             idx_v, rows_a, rows_b, pooled_v, sem_a, sem_b):
        wid = lax.axis_index("s") * _NC + lax.axis_index("c")
        tok_base = pl.multiple_of(wid * (SPW * T), 8)
        pltpu.sync_copy(toks_hbm.at[pl.ds(tok_base, SPW * T)], idx_v)

        def copies(s, rows, sem):
            res = []
            for (o, c) in chunks:
                src = table_hbm.at[idx_v.at[pl.ds(pl.multiple_of(s * T + o, 8), c)]]
                res.append(pltpu.make_async_copy(src, rows.at[pl.ds(o, c)], sem))
            return res

        def start(s, rows, sem):
            for cp in copies(s, rows, sem):
                cp.start()

        def wait(s, rows, sem):
            for cp in copies(s, rows, sem):
                cp.wait()

        def reduce_store(s, rows):
            ngrp = EMB // _LANES

            def rbody(t, acc):
                return tuple(acc[g] + rows[t, pl.ds(g * _LANES, _LANES)]
                             for g in range(ngrp))

            zero = tuple(jnp.zeros((_LANES,), jnp.float32) for _ in range(ngrp))
            acc = lax.fori_loop(0, T, rbody, zero, unroll=8)
            for g in range(ngrp):
                pooled_v[s, pl.ds(g * _LANES, _LANES)] = acc[g]

        start(0, rows_a, sem_a)

        def pair(p, carry):
            s0 = p * 2
            start(s0 + 1, rows_b, sem_b)
            wait(s0, rows_a, sem_a)
            reduce_store(s0, rows_a)

            @pl.when(s0 + 2 < SPW)
            def _():
                start(s0 + 2, rows_a, sem_a)

            wait(s0 + 1, rows_b, sem_b)
            reduce_store(s0 + 1, rows_b)
            return carry

        lax.fori_loop(0, SPW // 2, pair, 0)

        out_base = pl.multiple_of(wid * SPW, 8)
        pltpu.sync_copy(pooled_v, out_hbm.at[pl.ds(out_base, SPW)])

    return pl.kernel(
        body,
        out_type=jax.ShapeDtypeStruct((N, EMB), jnp.float32),
        mesh=mesh,
        compiler_params=pltpu.CompilerParams(use_tc_tiling_on_sc=False),
        scratch_types=[
            pltpu.VMEM((SPW * T,), jnp.int32),
            pltpu.VMEM((T, EMB), jnp.float32),
            pltpu.VMEM((T, EMB), jnp.float32),
            pltpu.VMEM((SPW, EMB), jnp.float32),
            pltpu.SemaphoreType.DMA,
            pltpu.SemaphoreType.DMA,
        ],
    )


def _head_body(ps_ref, w_ref, b_ref, stdev_ref, noise_ref, out_ref, *, T):
    pooled = ps_ref[...] * (1.0 / T)
    logits = jnp.dot(pooled, w_ref[...], preferred_element_type=jnp.float32)
    logits = logits + b_ref[...] + stdev_ref[0, 0] * noise_ref[...]
    out_ref[...] = jax.nn.log_softmax(logits, axis=-1)


def kernel(toks, tok_emb, W, b, stdev, noise):
    N, T = toks.shape
    V, EMB = tok_emb.shape
    NCLS = W.shape[1]
    NT = N * T
    toks_flat = toks.reshape(-1).astype(jnp.int32)
    table_p, idx_p = _make_relayout(V, EMB, NT)(tok_emb.T, toks_flat)
    vrows = table_p.shape[0] * 4
    table_lin = table_p.reshape(vrows, EMB)
    pooled_sum = _make_pooling_kernel(N, T, EMB, vrows)(idx_p[:NT], table_lin)
    head = pl.pallas_call(
        functools.partial(_head_body, T=T),
        out_shape=jax.ShapeDtypeStruct((N, NCLS), jnp.float32),
    )
    return head(pooled_sum, W, b.reshape(1, NCLS),
                stdev.reshape(1, 1), noise)


# speedup vs baseline: 3.0953x; 1.2055x over previous
import functools

import jax
import jax.numpy as jnp
from jax import lax
from jax.experimental import pallas as pl
from jax.experimental.pallas import tpu as pltpu
from jax.experimental.pallas import tpu_sc as plsc

_NC = 2
_NS = 16
_LANES = 16
_MAX_IDX = 128
_CHUNK = 2048


def _relayout_body(src_ref, toks_ref, out_ref, idx_ref, *, EMB):
    x = src_ref[...]
    G = _CHUNK // 4
    x4 = jnp.concatenate([x[:, G * q:G * (q + 1)] for q in range(4)], axis=0)
    out_ref[...] = jnp.swapaxes(x4, 0, 1)
    t = toks_ref[...]
    rem = jnp.bitwise_and(t, _CHUNK - 1)
    g = (jnp.bitwise_and(t, jnp.int32(~(_CHUNK - 1))) +
         (jnp.bitwise_and(rem, G - 1) << 2) + (rem // G))
    idx_ref[...] = g


def _make_relayout(V, EMB, NT):
    grid = (V + _CHUNK - 1) // _CHUNK
    rows_out = grid * (_CHUNK // 4)
    assert grid * _CHUNK >= NT
    return pl.pallas_call(
        functools.partial(_relayout_body, EMB=EMB),
        grid=(grid,),
        in_specs=[
            pl.BlockSpec((EMB, _CHUNK), lambda i: (0, i)),
            pl.BlockSpec((_CHUNK,), lambda i: (i,)),
        ],
        out_specs=[
            pl.BlockSpec((_CHUNK // 4, 4 * EMB), lambda i: (i, 0)),
            pl.BlockSpec((_CHUNK,), lambda i: (i,)),
        ],
        out_shape=[
            jax.ShapeDtypeStruct((rows_out, 4 * EMB), jnp.float32),
            jax.ShapeDtypeStruct((grid * _CHUNK,), jnp.int32),
        ],
    )


def _make_pooling_kernel(N, T, EMB, VROWS):
    NW = _NC * _NS
    assert N % NW == 0, N
    SPW = N // NW
    assert SPW % 2 == 0, SPW
    assert EMB % _LANES == 0, EMB
    assert T % 8 == 0, T

    chunks = []
    off = 0
    while off < T:
        c = min(_MAX_IDX, T - off)
        chunks.append((off, c))
        off += c

    mesh = plsc.VectorSubcoreMesh(
        core_axis_name="c", subcore_axis_name="s",
        num_cores=_NC, num_subcores=_NS)

    def body(toks_hbm, table_hbm, out_hbm,
             idx_v, rows_a, rows_b, pooled_v, sem_a, sem_b):
        wid = lax.axis_index("s") * _NC + lax.axis_index("c")
        tok_base = pl.multiple_of(wid * (SPW * T), 8)
        pltpu.sync_copy(toks_hbm.at[pl.ds(tok_base, SPW * T)], idx_v)

        def copies(s, rows, sem):
            res = []
            for (o, c) in chunks:
                src = table_hbm.at[idx_v.at[pl.ds(pl.multiple_of(s * T + o, 8), c)]]
                res.append(pltpu.make_async_copy(src, rows.at[pl.ds(o, c)], sem))
            return res

        def start(s, rows, sem):
            for cp in copies(s, rows, sem):
                cp.start()

        def wait(s, rows, sem):
            for cp in copies(s, rows, sem):
                cp.wait()

        def reduce_store(s, rows):
            ngrp = EMB // _LANES

            def rbody(t, acc):
                return tuple(acc[g] + rows[t, pl.ds(g * _LANES, _LANES)]
                             for g in range(ngrp))

            zero = tuple(jnp.zeros((_LANES,), jnp.float32) for _ in range(ngrp))
            acc = lax.fori_loop(0, T, rbody, zero, unroll=8)
            for g in range(ngrp):
                pooled_v[s, pl.ds(g * _LANES, _LANES)] = acc[g]

        start(0, rows_a, sem_a)

        def pair(p, carry):
            s0 = p * 2
            start(s0 + 1, rows_b, sem_b)
            wait(s0, rows_a, sem_a)
            reduce_store(s0, rows_a)

            @pl.when(s0 + 2 < SPW)
            def _():
                start(s0 + 2, rows_a, sem_a)

            wait(s0 + 1, rows_b, sem_b)
            reduce_store(s0 + 1, rows_b)
            return carry

        lax.fori_loop(0, SPW // 2, pair, 0)

        out_base = pl.multiple_of(wid * SPW, 8)
        pltpu.sync_copy(pooled_v, out_hbm.at[pl.ds(out_base, SPW)])

    return pl.kernel(
        body,
        out_type=jax.ShapeDtypeStruct((N, EMB), jnp.float32),
        mesh=mesh,
        compiler_params=pltpu.CompilerParams(use_tc_tiling_on_sc=False),
        scratch_types=[
            pltpu.VMEM((SPW * T,), jnp.int32),
            pltpu.VMEM((T, EMB), jnp.float32),
            pltpu.VMEM((T, EMB), jnp.float32),
            pltpu.VMEM((SPW, EMB), jnp.float32),
            pltpu.SemaphoreType.DMA,
            pltpu.SemaphoreType.DMA,
        ],
    )


def _head_body(ps_ref, w_ref, b_ref, stdev_ref, noise_ref, out_ref, *, T):
    pooled = ps_ref[...] * (1.0 / T)
    logits = jnp.dot(pooled, w_ref[...], preferred_element_type=jnp.float32)
    logits = logits + b_ref[...] + stdev_ref[0, 0] * noise_ref[...]
    out_ref[...] = jax.nn.log_softmax(logits, axis=-1)


def kernel(toks, tok_emb, W, b, stdev, noise):
    N, T = toks.shape
    V, EMB = tok_emb.shape
    NCLS = W.shape[1]
    NT = N * T
    toks_flat = toks.reshape(-1).astype(jnp.int32)
    table_p, idx_p = _make_relayout(V, EMB, NT)(tok_emb.T, toks_flat)
    vrows = table_p.shape[0] * 4
    table_lin = table_p.reshape(vrows, EMB)
    pooled_sum = _make_pooling_kernel(N, T, EMB, vrows)(idx_p[:NT], table_lin)
    head = pl.pallas_call(
        functools.partial(_head_body, T=T),
        out_shape=jax.ShapeDtypeStruct((N, NCLS), jnp.float32),
    )
    return head(pooled_sum, W, b.reshape(1, NCLS),
                stdev.reshape(1, 1), noise)


# speedup vs baseline: 5.2818x; 1.7064x over previous
import functools

import jax
import jax.numpy as jnp
from jax import lax
from jax.experimental import pallas as pl
from jax.experimental.pallas import tpu as pltpu
from jax.experimental.pallas import tpu_sc as plsc

_NC = 2
_NS = 16
_LANES = 16
_MAX_IDX = 128
_CHUNK = 8192


def _relayout_body(src_ref, toks_ref, out_ref, idx_ref, *, EMB):
    x = src_ref[...]
    G = _CHUNK // 4
    x4 = jnp.concatenate([x[:, G * q:G * (q + 1)] for q in range(4)], axis=0)
    out_ref[...] = jnp.swapaxes(x4, 0, 1)
    t = toks_ref[...]
    rem = jnp.bitwise_and(t, _CHUNK - 1)
    g = (jnp.bitwise_and(t, jnp.int32(~(_CHUNK - 1))) +
         (jnp.bitwise_and(rem, G - 1) << 2) + (rem // G))
    idx_ref[...] = g


def _make_relayout(V, EMB, NT):
    grid = (V + _CHUNK - 1) // _CHUNK
    rows_out = grid * (_CHUNK // 4)
    assert grid * _CHUNK >= NT
    return pl.pallas_call(
        functools.partial(_relayout_body, EMB=EMB),
        grid=(grid,),
        in_specs=[
            pl.BlockSpec((EMB, _CHUNK), lambda i: (0, i)),
            pl.BlockSpec((_CHUNK,), lambda i: (i,)),
        ],
        out_specs=[
            pl.BlockSpec((_CHUNK // 4, 4 * EMB), lambda i: (i, 0)),
            pl.BlockSpec((_CHUNK,), lambda i: (i,)),
        ],
        out_shape=[
            jax.ShapeDtypeStruct((rows_out, 4 * EMB), jnp.float32),
            jax.ShapeDtypeStruct((grid * _CHUNK,), jnp.int32),
        ],
    )


def _make_pooling_kernel(N, T, EMB, VROWS):
    NW = _NC * _NS
    assert N % NW == 0, N
    SPW = N // NW
    assert SPW % 2 == 0, SPW
    assert EMB % _LANES == 0, EMB
    assert T % 8 == 0, T

    chunks = []
    off = 0
    while off < T:
        c = min(_MAX_IDX, T - off)
        chunks.append((off, c))
        off += c

    mesh = plsc.VectorSubcoreMesh(
        core_axis_name="c", subcore_axis_name="s",
        num_cores=_NC, num_subcores=_NS)

    def body(toks_hbm, table_hbm, out_hbm,
             idx_v, rows_a, rows_b, pooled_v, sem_a, sem_b):
        wid = lax.axis_index("s") * _NC + lax.axis_index("c")
        tok_base = pl.multiple_of(wid * (SPW * T), 8)
        pltpu.sync_copy(toks_hbm.at[pl.ds(tok_base, SPW * T)], idx_v)

        def copies(s, rows, sem):
            res = []
            for (o, c) in chunks:
                src = table_hbm.at[idx_v.at[pl.ds(pl.multiple_of(s * T + o, 8), c)]]
                res.append(pltpu.make_async_copy(src, rows.at[pl.ds(o, c)], sem))
            return res

        def start(s, rows, sem):
            for cp in copies(s, rows, sem):
                cp.start()

        def wait(s, rows, sem):
            for cp in copies(s, rows, sem):
                cp.wait()

        def reduce_store(s, rows):
            ngrp = EMB // _LANES

            def rbody(t, acc):
                return tuple(acc[g] + rows[t, pl.ds(g * _LANES, _LANES)]
                             for g in range(ngrp))

            zero = tuple(jnp.zeros((_LANES,), jnp.float32) for _ in range(ngrp))
            acc = lax.fori_loop(0, T, rbody, zero, unroll=8)
            for g in range(ngrp):
                pooled_v[s, pl.ds(g * _LANES, _LANES)] = acc[g]

        start(0, rows_a, sem_a)

        def pair(p, carry):
            s0 = p * 2
            start(s0 + 1, rows_b, sem_b)
            wait(s0, rows_a, sem_a)
            reduce_store(s0, rows_a)

            @pl.when(s0 + 2 < SPW)
            def _():
                start(s0 + 2, rows_a, sem_a)

            wait(s0 + 1, rows_b, sem_b)
            reduce_store(s0 + 1, rows_b)
            return carry

        lax.fori_loop(0, SPW // 2, pair, 0)

        out_base = pl.multiple_of(wid * SPW, 8)
        pltpu.sync_copy(pooled_v, out_hbm.at[pl.ds(out_base, SPW)])

    return pl.kernel(
        body,
        out_type=jax.ShapeDtypeStruct((N, EMB), jnp.float32),
        mesh=mesh,
        compiler_params=pltpu.CompilerParams(use_tc_tiling_on_sc=False),
        scratch_types=[
            pltpu.VMEM((SPW * T,), jnp.int32),
            pltpu.VMEM((T, EMB), jnp.float32),
            pltpu.VMEM((T, EMB), jnp.float32),
            pltpu.VMEM((SPW, EMB), jnp.float32),
            pltpu.SemaphoreType.DMA,
            pltpu.SemaphoreType.DMA,
        ],
    )


def _head_body(ps_ref, w_ref, b_ref, stdev_ref, noise_ref, out_ref, *, T):
    pooled = ps_ref[...] * (1.0 / T)
    logits = jnp.dot(pooled, w_ref[...], preferred_element_type=jnp.float32)
    logits = logits + b_ref[...] + stdev_ref[0, 0] * noise_ref[...]
    out_ref[...] = jax.nn.log_softmax(logits, axis=-1)


def kernel(toks, tok_emb, W, b, stdev, noise):
    N, T = toks.shape
    V, EMB = tok_emb.shape
    NCLS = W.shape[1]
    NT = N * T
    toks_flat = toks.reshape(-1).astype(jnp.int32)
    table_p, idx_p = _make_relayout(V, EMB, NT)(tok_emb.T, toks_flat)
    vrows = table_p.shape[0] * 4
    table_lin = table_p.reshape(vrows, EMB)
    pooled_sum = _make_pooling_kernel(N, T, EMB, vrows)(idx_p[:NT], table_lin)
    head = pl.pallas_call(
        functools.partial(_head_body, T=T),
        out_shape=jax.ShapeDtypeStruct((N, NCLS), jnp.float32),
    )
    return head(pooled_sum, W, b.reshape(1, NCLS),
                stdev.reshape(1, 1), noise)


# speedup vs baseline: 6.6230x; 1.2539x over previous
import functools

import jax
import jax.numpy as jnp
from jax import lax
from jax.experimental import pallas as pl
from jax.experimental.pallas import tpu as pltpu
from jax.experimental.pallas import tpu_sc as plsc

_NC = 2
_NS = 16
_LANES = 16
_MAX_IDX = 128
_CHUNK = 32768


def _relayout_body(src_ref, toks_ref, out_ref, idx_ref, *, EMB):
    x = src_ref[...]
    G = _CHUNK // 4
    x4 = jnp.concatenate([x[:, G * q:G * (q + 1)] for q in range(4)], axis=0)
    out_ref[...] = jnp.swapaxes(x4, 0, 1)
    t = toks_ref[...]
    rem = jnp.bitwise_and(t, _CHUNK - 1)
    g = (jnp.bitwise_and(t, jnp.int32(~(_CHUNK - 1))) +
         (jnp.bitwise_and(rem, G - 1) << 2) + (rem // G))
    idx_ref[...] = g


def _make_relayout(V, EMB, NT):
    grid = (V + _CHUNK - 1) // _CHUNK
    rows_out = grid * (_CHUNK // 4)
    assert grid * _CHUNK >= NT
    return pl.pallas_call(
        functools.partial(_relayout_body, EMB=EMB),
        grid=(grid,),
        in_specs=[
            pl.BlockSpec((EMB, _CHUNK), lambda i: (0, i)),
            pl.BlockSpec((_CHUNK,), lambda i: (i,)),
        ],
        out_specs=[
            pl.BlockSpec((_CHUNK // 4, 4 * EMB), lambda i: (i, 0)),
            pl.BlockSpec((_CHUNK,), lambda i: (i,)),
        ],
        out_shape=[
            jax.ShapeDtypeStruct((rows_out, 4 * EMB), jnp.float32),
            jax.ShapeDtypeStruct((grid * _CHUNK,), jnp.int32),
        ],
    )


def _make_pooling_kernel(N, T, EMB, VROWS):
    NW = _NC * _NS
    assert N % NW == 0, N
    SPW = N // NW
    assert SPW % 2 == 0, SPW
    assert EMB % _LANES == 0, EMB
    assert T % 8 == 0, T

    chunks = []
    off = 0
    while off < T:
        c = min(_MAX_IDX, T - off)
        chunks.append((off, c))
        off += c

    mesh = plsc.VectorSubcoreMesh(
        core_axis_name="c", subcore_axis_name="s",
        num_cores=_NC, num_subcores=_NS)

    def body(toks_hbm, table_hbm, out_hbm,
             idx_v, rows_a, rows_b, pooled_v, sem_a, sem_b):
        wid = lax.axis_index("s") * _NC + lax.axis_index("c")
        tok_base = pl.multiple_of(wid * (SPW * T), 8)
        pltpu.sync_copy(toks_hbm.at[pl.ds(tok_base, SPW * T)], idx_v)

        def copies(s, rows, sem):
            res = []
            for (o, c) in chunks:
                src = table_hbm.at[idx_v.at[pl.ds(pl.multiple_of(s * T + o, 8), c)]]
                res.append(pltpu.make_async_copy(src, rows.at[pl.ds(o, c)], sem))
            return res

        def start(s, rows, sem):
            for cp in copies(s, rows, sem):
                cp.start()

        def wait(s, rows, sem):
            for cp in copies(s, rows, sem):
                cp.wait()

        def reduce_store(s, rows):
            ngrp = EMB // _LANES

            def rbody(t, acc):
                return tuple(acc[g] + rows[t, pl.ds(g * _LANES, _LANES)]
                             for g in range(ngrp))

            zero = tuple(jnp.zeros((_LANES,), jnp.float32) for _ in range(ngrp))
            acc = lax.fori_loop(0, T, rbody, zero, unroll=8)
            for g in range(ngrp):
                pooled_v[s, pl.ds(g * _LANES, _LANES)] = acc[g]

        start(0, rows_a, sem_a)

        def pair(p, carry):
            s0 = p * 2
            start(s0 + 1, rows_b, sem_b)
            wait(s0, rows_a, sem_a)
            reduce_store(s0, rows_a)

            @pl.when(s0 + 2 < SPW)
            def _():
                start(s0 + 2, rows_a, sem_a)

            wait(s0 + 1, rows_b, sem_b)
            reduce_store(s0 + 1, rows_b)
            return carry

        lax.fori_loop(0, SPW // 2, pair, 0)

        out_base = pl.multiple_of(wid * SPW, 8)
        pltpu.sync_copy(pooled_v, out_hbm.at[pl.ds(out_base, SPW)])

    return pl.kernel(
        body,
        out_type=jax.ShapeDtypeStruct((N, EMB), jnp.float32),
        mesh=mesh,
        compiler_params=pltpu.CompilerParams(use_tc_tiling_on_sc=False),
        scratch_types=[
            pltpu.VMEM((SPW * T,), jnp.int32),
            pltpu.VMEM((T, EMB), jnp.float32),
            pltpu.VMEM((T, EMB), jnp.float32),
            pltpu.VMEM((SPW, EMB), jnp.float32),
            pltpu.SemaphoreType.DMA,
            pltpu.SemaphoreType.DMA,
        ],
    )


def _head_body(ps_ref, w_ref, b_ref, stdev_ref, noise_ref, out_ref, *, T):
    pooled = ps_ref[...] * (1.0 / T)
    logits = jnp.dot(pooled, w_ref[...], preferred_element_type=jnp.float32)
    logits = logits + b_ref[...] + stdev_ref[0, 0] * noise_ref[...]
    out_ref[...] = jax.nn.log_softmax(logits, axis=-1)


def kernel(toks, tok_emb, W, b, stdev, noise):
    N, T = toks.shape
    V, EMB = tok_emb.shape
    NCLS = W.shape[1]
    NT = N * T
    toks_flat = toks.reshape(-1).astype(jnp.int32)
    table_p, idx_p = _make_relayout(V, EMB, NT)(tok_emb.T, toks_flat)
    vrows = table_p.shape[0] * 4
    table_lin = table_p.reshape(vrows, EMB)
    pooled_sum = _make_pooling_kernel(N, T, EMB, vrows)(idx_p[:NT], table_lin)
    head = pl.pallas_call(
        functools.partial(_head_body, T=T),
        out_shape=jax.ShapeDtypeStruct((N, NCLS), jnp.float32),
    )
    return head(pooled_sum, W, b.reshape(1, NCLS),
                stdev.reshape(1, 1), noise)
